# Initial kernel scaffold; baseline (speedup 1.0000x reference)
import jax
import jax.numpy as jnp
from jax.experimental import pallas as pl


def kernel(x, edge_attr, params, edge_index, batch, assignment_index_2, edge_index_2, batch_2):
    raise NotImplementedError("write your pallas kernel here")


# speedup vs baseline: 2.3025x; 2.3025x over previous
import jax
import jax.numpy as jnp
from jax import lax
from jax.experimental import pallas as pl
from jax.experimental.pallas import tpu as pltpu
from jax.experimental.pallas import tpu_sc as plsc

_CH = 128
_RCH = 128


def _mm(x, w, b=None, act=None, block=2000):
    n, k = x.shape
    m = w.shape[1]
    nb = n // block
    has_b = b is not None

    def body(*refs):
        if has_b:
            x_ref, w_ref, b_ref, o_ref = refs
        else:
            x_ref, w_ref, o_ref = refs
        y = jnp.dot(x_ref[...], w_ref[...], preferred_element_type=jnp.float32)
        if has_b:
            y = y + b_ref[...]
        if act == "sigmoid":
            y = jax.nn.sigmoid(y)
        o_ref[...] = y

    in_specs = [pl.BlockSpec((block, k), lambda i: (i, 0)),
                pl.BlockSpec((k, m), lambda i: (0, 0))]
    args = [x, w]
    if has_b:
        in_specs.append(pl.BlockSpec((1, m), lambda i: (0, 0)))
        args.append(b)
    return pl.pallas_call(
        body, grid=(nb,), in_specs=in_specs,
        out_specs=pl.BlockSpec((block, m), lambda i: (i, 0)),
        out_shape=jax.ShapeDtypeStruct((n, m), jnp.float32),
    )(*args)


def _gru(agg, h, wihT, whhT, bih, bhh, relu_out, block):
    n, c = h.shape
    nb = n // block

    def body(a_ref, h_ref, wi_ref, wh_ref, bi_ref, bh_ref, o_ref):
        hh = h_ref[...]
        gi = jnp.dot(a_ref[...], wi_ref[...], preferred_element_type=jnp.float32) + bi_ref[...]
        gh = jnp.dot(hh, wh_ref[...], preferred_element_type=jnp.float32) + bh_ref[...]
        r = jax.nn.sigmoid(gi[:, :c] + gh[:, :c])
        z = jax.nn.sigmoid(gi[:, c:2 * c] + gh[:, c:2 * c])
        nn_ = jnp.tanh(gi[:, 2 * c:] + r * gh[:, 2 * c:])
        out = (1.0 - z) * nn_ + z * hh
        if relu_out:
            out = jnp.maximum(out, 0.0)
        o_ref[...] = out

    return pl.pallas_call(
        body, grid=(nb,),
        in_specs=[pl.BlockSpec((block, c), lambda i: (i, 0)),
                  pl.BlockSpec((block, c), lambda i: (i, 0)),
                  pl.BlockSpec((c, 3 * c), lambda i: (0, 0)),
                  pl.BlockSpec((c, 3 * c), lambda i: (0, 0)),
                  pl.BlockSpec((1, 3 * c), lambda i: (0, 0)),
                  pl.BlockSpec((1, 3 * c), lambda i: (0, 0))],
        out_specs=pl.BlockSpec((block, c), lambda i: (i, 0)),
        out_shape=jax.ShapeDtypeStruct((n, c), jnp.float32),
    )(agg, h, wihT, whhT, bih, bhh)


def _col_reduce(h, block, s_mu=None):
    n, c = h.shape
    nb = n // block
    inv_n = 1.0 / n

    def body(*refs):
        if s_mu is None:
            h_ref, o_ref = refs
            cur = jnp.sum(h_ref[...], axis=0, keepdims=True)
        else:
            h_ref, m_ref, o_ref = refs
            d = h_ref[...] - m_ref[0:1, :] * inv_n
            cur = jnp.sum(d * d, axis=0, keepdims=True)
        cur = jnp.concatenate([cur, jnp.zeros((7, c), jnp.float32)], axis=0)

        @pl.when(pl.program_id(0) == 0)
        def _():
            o_ref[...] = jnp.zeros((8, c), jnp.float32)

        o_ref[...] += cur

    in_specs = [pl.BlockSpec((block, c), lambda i: (i, 0))]
    args = [h]
    if s_mu is not None:
        in_specs.append(pl.BlockSpec((8, c), lambda i: (0, 0)))
        args.append(s_mu)
    return pl.pallas_call(
        body, grid=(nb,),
        in_specs=in_specs,
        out_specs=pl.BlockSpec((8, c), lambda i: (0, 0)),
        out_shape=jax.ShapeDtypeStruct((8, c), jnp.float32),
    )(*args)


def _bn_apply_relu(h, s1, s2, gamma, beta, block):
    n, c = h.shape
    nb = n // block
    inv_n = 1.0 / n

    def body(h_ref, s1_ref, s2_ref, g_ref, b_ref, o_ref):
        mu = s1_ref[0:1, :] * inv_n
        var = s2_ref[0:1, :] * inv_n
        y = (h_ref[...] - mu) / jnp.sqrt(var + 1e-5) * g_ref[...] + b_ref[...]
        o_ref[...] = jnp.maximum(y, 0.0)

    return pl.pallas_call(
        body, grid=(nb,),
        in_specs=[pl.BlockSpec((block, c), lambda i: (i, 0)),
                  pl.BlockSpec((8, c), lambda i: (0, 0)),
                  pl.BlockSpec((8, c), lambda i: (0, 0)),
                  pl.BlockSpec((1, c), lambda i: (0, 0)),
                  pl.BlockSpec((1, c), lambda i: (0, 0))],
        out_specs=pl.BlockSpec((block, c), lambda i: (i, 0)),
        out_shape=jax.ShapeDtypeStruct((n, c), jnp.float32),
    )(h, s1, s2, gamma, beta)


def _pool_div(sums, cnt, block):
    n, c = sums.shape
    nb = n // block

    def body(s_ref, c_ref, o_ref):
        cc = jnp.maximum(c_ref[...][:, 0:1], 1.0)
        o_ref[...] = s_ref[...] / cc

    return pl.pallas_call(
        body, grid=(nb,),
        in_specs=[pl.BlockSpec((block, c), lambda i: (i, 0)),
                  pl.BlockSpec((block, 16), lambda i: (i, 0))],
        out_specs=pl.BlockSpec((block, c), lambda i: (i, 0)),
        out_shape=jax.ShapeDtypeStruct((n, c), jnp.float32),
    )(sums, cnt)


_NEG = -3.0e38


def _s2s_pass1(x, batch3, q, block):
    n, c = x.shape
    g = q.shape[0]
    nb = n // block

    def body(x_ref, b_ref, q_ref, o_ref):
        ids = b_ref[...].reshape(block, 1)
        gi = lax.broadcasted_iota(jnp.int32, (1, g), 1)
        oh = ids == gi
        ohf = oh.astype(jnp.float32)
        qb = jnp.dot(ohf, q_ref[...], preferred_element_type=jnp.float32,
                     precision=lax.Precision.HIGHEST)
        e = jnp.sum(x_ref[...] * qb, axis=1, keepdims=True)
        mcur = jnp.max(jnp.where(oh, jnp.broadcast_to(e, (block, g)), _NEG),
                       axis=0, keepdims=True)
        cur = jnp.broadcast_to(mcur, (8, g))

        @pl.when(pl.program_id(0) == 0)
        def _():
            o_ref[...] = jnp.full((8, g), _NEG, jnp.float32)

        o_ref[...] = jnp.maximum(o_ref[...], cur)

    return pl.pallas_call(
        body, grid=(nb,),
        in_specs=[pl.BlockSpec((block, c), lambda i: (i, 0)),
                  pl.BlockSpec((1, block, 1), lambda i: (i, 0, 0)),
                  pl.BlockSpec((g, c), lambda i: (0, 0))],
        out_specs=pl.BlockSpec((8, g), lambda i: (0, 0)),
        out_shape=jax.ShapeDtypeStruct((8, g), jnp.float32),
    )(x, batch3, q)


def _s2s_pass2(x, batch3, q, mstats, block):
    n, c = x.shape
    g = q.shape[0]
    nb = n // block

    def body(x_ref, b_ref, q_ref, m_ref, o_ref):
        ids = b_ref[...].reshape(block, 1)
        gi = lax.broadcasted_iota(jnp.int32, (1, g), 1)
        ohf = (ids == gi).astype(jnp.float32)
        qb = jnp.dot(ohf, q_ref[...], preferred_element_type=jnp.float32,
                     precision=lax.Precision.HIGHEST)
        e = jnp.sum(x_ref[...] * qb, axis=1, keepdims=True)
        ms = m_ref[...]
        ms = jnp.where(ms < -1.0e37, 0.0, ms)
        mb = lax.dot_general(ohf, ms, (((1,), (1,)), ((), ())),
                             preferred_element_type=jnp.float32,
                             precision=lax.Precision.HIGHEST)
        ex = jnp.exp(e - mb[:, 0:1])
        rhs = jnp.concatenate([x_ref[...] * ex, jnp.broadcast_to(ex, (block, 8))],
                              axis=1)
        cur = lax.dot_general(ohf, rhs, (((0,), (0,)), ((), ())),
                              preferred_element_type=jnp.float32,
                              precision=lax.Precision.HIGHEST)

        @pl.when(pl.program_id(0) == 0)
        def _():
            o_ref[...] = jnp.zeros((g, c + 8), jnp.float32)

        o_ref[...] += cur

    return pl.pallas_call(
        body, grid=(nb,),
        in_specs=[pl.BlockSpec((block, c), lambda i: (i, 0)),
                  pl.BlockSpec((1, block, 1), lambda i: (i, 0, 0)),
                  pl.BlockSpec((g, c), lambda i: (0, 0)),
                  pl.BlockSpec((8, g), lambda i: (0, 0))],
        out_specs=pl.BlockSpec((g, c + 8), lambda i: (0, 0)),
        out_shape=jax.ShapeDtypeStruct((g, c + 8), jnp.float32),
    )(x, batch3, q, mstats)


def _s2s_lstm(us, q_prev, h0, c0, h1, c1, w):
    g, c = q_prev.shape
    wi0T, wh0T, bi0, bh0, wi1T, wh1T, bi1, bh1 = w

    def body(us_ref, qp_ref, h0_ref, c0_ref, h1_ref, c1_ref,
             wi0_ref, wh0_ref, bi0_ref, bh0_ref,
             wi1_ref, wh1_ref, bi1_ref, bh1_ref,
             qo, h0o, c0o, h1o, c1o):
        usv = us_ref[...]
        r = usv[:, :c] / (usv[:, c:c + 1] + 1e-16)
        qstar = jnp.concatenate([qp_ref[...], r], axis=1)

        def cell(inp, hr, cr, wi, wh, bi, bh):
            gg = (jnp.dot(inp, wi, preferred_element_type=jnp.float32) + bi
                  + jnp.dot(hr, wh, preferred_element_type=jnp.float32) + bh)
            i = jax.nn.sigmoid(gg[:, :c])
            f = jax.nn.sigmoid(gg[:, c:2 * c])
            gt = jnp.tanh(gg[:, 2 * c:3 * c])
            o = jax.nn.sigmoid(gg[:, 3 * c:])
            cn = f * cr + i * gt
            return o * jnp.tanh(cn), cn

        h0n, c0n = cell(qstar, h0_ref[...], c0_ref[...],
                        wi0_ref[...], wh0_ref[...], bi0_ref[...], bh0_ref[...])
        h1n, c1n = cell(h0n, h1_ref[...], c1_ref[...],
                        wi1_ref[...], wh1_ref[...], bi1_ref[...], bh1_ref[...])
        qo[...] = h1n
        h0o[...] = h0n
        c0o[...] = c0n
        h1o[...] = h1n
        c1o[...] = c1n

    full = lambda a: pl.BlockSpec(a.shape, lambda: tuple(0 for _ in a.shape))
    args = [us, q_prev, h0, c0, h1, c1, wi0T, wh0T, bi0, bh0, wi1T, wh1T, bi1, bh1]
    outs = [jax.ShapeDtypeStruct((g, c), jnp.float32)] * 5
    return pl.pallas_call(
        body, grid=(),
        in_specs=[full(a) for a in args],
        out_specs=[pl.BlockSpec((g, c), lambda: (0, 0))] * 5,
        out_shape=outs,
    )(*args)


def _head(q1, us1, q2, us2, pg, pb, w0, b0, g0, bb0, w1, b1, g1, bb1, w2, b2):
    g, c = q1.shape

    def body(q1r, us1r, q2r, us2r, pgr, pbr, w0r, b0r, g0r, bb0r,
             w1r, b1r, g1r, bb1r, w2r, b2r, o_ref):
        def bn(v, gm, bt):
            mu = jnp.mean(v, axis=0, keepdims=True)
            d = v - mu
            var = jnp.mean(d * d, axis=0, keepdims=True)
            return d / jnp.sqrt(var + 1e-5) * gm + bt

        def xcat(qr, usr):
            usv = usr[...]
            r = usv[:, :c] / (usv[:, c:c + 1] + 1e-16)
            return jnp.concatenate([qr[...], r], axis=1)

        z = jnp.concatenate([xcat(q1r, us1r), xcat(q2r, us2r)], axis=1)
        z = bn(z, pgr[...], pbr[...])
        z = jnp.dot(z, w0r[...], preferred_element_type=jnp.float32) + b0r[...]
        z = jnp.maximum(bn(z, g0r[...], bb0r[...]), 0.0)
        z = jnp.dot(z, w1r[...], preferred_element_type=jnp.float32) + b1r[...]
        z = jnp.maximum(bn(z, g1r[...], bb1r[...]), 0.0)
        o_ref[...] = jnp.dot(z, w2r[...], preferred_element_type=jnp.float32) + b2r[...]

    args = [q1, us1, q2, us2, pg, pb, w0, b0, g0, bb0, w1, b1, g1, bb1, w2, b2]
    full = lambda a: pl.BlockSpec(a.shape, lambda: tuple(0 for _ in a.shape))
    return pl.pallas_call(
        body, grid=(),
        in_specs=[full(a) for a in args],
        out_specs=pl.BlockSpec((g, 1), lambda: (0, 0)),
        out_shape=jax.ShapeDtypeStruct((g, 1), jnp.float32),
    )(*args)


def _edge_plan(ei, n_nodes):
    src = ei[0].astype(jnp.int32)
    dst = ei[1].astype(jnp.int32)
    order = jnp.argsort(dst, stable=True).astype(jnp.int32)
    dst_s = jnp.take(dst, order)
    src_s = jnp.take(src, order)
    w = jnp.arange(33, dtype=jnp.int32)
    bounds = (n_nodes * w) // 32
    off = jnp.searchsorted(dst_s, bounds).astype(jnp.int32)
    offs = jnp.zeros((32, 16), jnp.int32)
    offs = offs.at[:, 0].set(off[:32])
    offs = offs.at[:, 1].set(off[1:])
    offs = offs.at[:, 2].set(bounds[:32])
    offs = offs.at[:, 3].set(bounds[1:])
    pad = jnp.zeros((_CH,), jnp.int32)
    return (jnp.concatenate([src_s, pad]), jnp.concatenate([order, pad]),
            jnp.concatenate([dst_s, pad]), offs)


def _sc_round(m_arr, gate, plan, n_nodes, with_counts=False):
    src_p, perm_p, dst_p, offs = plan
    rows_base = -(-n_nodes // 32)
    dummy = rows_base
    t_rows = rows_base + 8
    gated = gate is not None
    mesh = plsc.VectorSubcoreMesh(core_axis_name="c", subcore_axis_name="s")

    def body(*refs):
        it = iter(refs)
        m_h = next(it)
        g_h = next(it) if gated else None
        src_h = next(it)
        perm_h = next(it) if gated else None
        dst_h = next(it)
        off_h = next(it)
        agg_o = next(it)
        cnt_o = next(it) if with_counts else None
        off_v = next(it)
        src_v = next(it)
        perm_v = next(it) if gated else None
        dst_v = next(it)
        mrows = next(it)
        grows = next(it) if gated else None
        acc = next(it)
        if with_counts:
            cacc = next(it)
        sem = next(it)
        sem2 = next(it) if gated else None

        c = lax.axis_index("c")
        s = lax.axis_index("s")
        i16 = lax.iota(jnp.int32, 16)
        zero16 = jnp.zeros((16,), jnp.float32)
        one16 = jnp.full((16,), 1.0, jnp.float32)

        def zrow(r, _):
            for k in range(4):
                acc[r, pl.ds(k * 16, 16)] = zero16
            if with_counts:
                cacc[r, pl.ds(0, 16)] = zero16
            return 0
        lax.fori_loop(0, t_rows, zrow, 0)

        pltpu.sync_copy(off_h, off_v)
        myrow = off_v[c * 16 + s]
        e_start = myrow[0]
        e_end = myrow[1]
        node_lo = myrow[2]
        width = myrow[3] - node_lo
        start8 = (e_start // 8) * 8
        nch = (e_end - start8 + _CH - 1) // _CH

        def chunk(i, _):
            base = pl.multiple_of(start8 + i * _CH, 8)
            pltpu.sync_copy(src_h.at[pl.ds(base, _CH)], src_v)
            if gated:
                pltpu.sync_copy(perm_h.at[pl.ds(base, _CH)], perm_v)
            pltpu.sync_copy(dst_h.at[pl.ds(base, _CH)], dst_v.at[pl.ds(0, _CH)])
            cp1 = pltpu.async_copy(m_h.at[src_v], mrows, sem)
            if gated:
                cp2 = pltpu.async_copy(g_h.at[perm_v], grows, sem2)
            for j in range(_CH // 16):
                sl = pl.ds(j * 16, 16)
                dv = dst_v[sl]
                pos = base + j * 16 + i16
                loc = dv - node_lo
                ok = (pos >= e_start) & (pos < e_end) & (loc >= 0) & (loc < width)
                dst_v[sl] = jnp.where(ok, loc, dummy)
            cp1.wait()
            if gated:
                cp2.wait()

            def edge_body(r, _):
                d = dst_v[pl.ds(r, 16)][0]
                for k in range(4):
                    sl2 = pl.ds(k * 16, 16)
                    if gated:
                        acc[d, sl2] = acc[d, sl2] + mrows[r, sl2] * grows[r, sl2]
                    else:
                        acc[d, sl2] = acc[d, sl2] + mrows[r, sl2]
                if with_counts:
                    cacc[d, pl.ds(0, 16)] = cacc[d, pl.ds(0, 16)] + one16
                return 0
            lax.fori_loop(0, _CH, edge_body, 0)
            return 0
        lax.fori_loop(0, nch, chunk, 0)

        nco = (width + _RCH - 1) // _RCH

        def ochunk(k, _):
            row = jnp.maximum(jnp.minimum(k * _RCH, width - _RCH), 0)
            pltpu.sync_copy(acc.at[pl.ds(row, _RCH)],
                            agg_o.at[pl.ds(node_lo + row, _RCH)])
            if with_counts:
                pltpu.sync_copy(cacc.at[pl.ds(row, _RCH)],
                                cnt_o.at[pl.ds(node_lo + row, _RCH)])
            return 0
        lax.fori_loop(0, nco, ochunk, 0)

    outs = [jax.ShapeDtypeStruct((n_nodes, 64), jnp.float32)]
    if with_counts:
        outs.append(jax.ShapeDtypeStruct((n_nodes, 16), jnp.float32))
    scratch = [pltpu.VMEM((32, 16), jnp.int32),
               pltpu.VMEM((_CH,), jnp.int32)]
    if gated:
        scratch.append(pltpu.VMEM((_CH,), jnp.int32))
    scratch.append(pltpu.VMEM((_CH + 16,), jnp.int32))
    scratch.append(pltpu.VMEM((_RCH, 64), jnp.float32))
    if gated:
        scratch.append(pltpu.VMEM((_CH, 64), jnp.float32))
    scratch.append(pltpu.VMEM((t_rows, 64), jnp.float32))
    if with_counts:
        scratch.append(pltpu.VMEM((t_rows, 16), jnp.float32))
    scratch.append(pltpu.SemaphoreType.DMA)
    if gated:
        scratch.append(pltpu.SemaphoreType.DMA)

    fn = pl.kernel(body, out_type=tuple(outs), mesh=mesh,
                   scratch_types=tuple(scratch),
                   compiler_params=pltpu.CompilerParams(use_tc_tiling_on_sc=False))
    args = [m_arr]
    if gated:
        args.append(gate)
    args.append(src_p)
    if gated:
        args.append(perm_p)
    args += [dst_p, offs]
    res = fn(*args)
    return res if with_counts else res[0]


def _mggc(h, plan, n_nodes, mp, gate, block):
    wihT = mp['Wih'].T
    whhT = mp['Whh'].T
    bih = mp['bih'].reshape(1, -1)
    bhh = mp['bhh'].reshape(1, -1)
    for l in range(mp['W'].shape[0]):
        m = _mm(h, mp['W'][l], block=block)
        agg = _sc_round(m, gate, plan, n_nodes)
        h = _gru(agg, h, wihT, whhT, bih, bhh, False, block)
    return h


def _set2set(xn, batch3, sp, block):
    g, c = 256, xn.shape[1]
    w = (sp['l0']['Wih'].T, sp['l0']['Whh'].T,
         sp['l0']['bih'].reshape(1, -1), sp['l0']['bhh'].reshape(1, -1),
         sp['l1']['Wih'].T, sp['l1']['Whh'].T,
         sp['l1']['bih'].reshape(1, -1), sp['l1']['bhh'].reshape(1, -1))
    q = jnp.zeros((g, c), jnp.float32)
    us = jnp.zeros((g, c + 8), jnp.float32)
    h0 = c0 = h1 = c1 = jnp.zeros((g, c), jnp.float32)
    for _ in range(5):
        q, h0, c0, h1, c1 = _s2s_lstm(us, q, h0, c0, h1, c1, w)
        mstats = _s2s_pass1(xn, batch3, q, block)
        us = _s2s_pass2(xn, batch3, q, mstats, block)
    return q, us


def kernel(x, edge_attr, params, edge_index, batch, assignment_index_2,
           edge_index_2, batch_2):
    p = params
    n, c = x.shape
    n2 = batch_2.shape[0]
    e = edge_index.shape[1]
    bn1 = 2000 if n % 2000 == 0 else n
    bn2 = 1000 if n2 % 1000 == 0 else n2
    be = 8000 if e % 8000 == 0 else e

    plan1 = _edge_plan(edge_index, n)
    plan2 = _edge_plan(edge_index_2, n2)
    plan_a = _edge_plan(assignment_index_2, n2)
    batch3 = batch.astype(jnp.int32).reshape(n // bn1, bn1, 1)
    batch3_2 = batch_2.astype(jnp.int32).reshape(n2 // bn2, bn2, 1)

    h = x
    for li, mp in enumerate([p['m1'], p['m2'], p['m3']]):
        gate = _mm(edge_attr, mp['We'], mp['be'].reshape(1, -1),
                   act="sigmoid", block=be)
        h = _mggc(h, plan1, n, mp, gate, bn1)
        s1 = _col_reduce(h, bn1)
        s2 = _col_reduce(h, bn1, s1)
        h = _bn_apply_relu(h, s1, s2, p['bn'][li][0].reshape(1, -1),
                           p['bn'][li][1].reshape(1, -1), bn1)

    q1, us1 = _set2set(h, batch3, p['s2s1'], bn1)

    sums, cnt = _sc_round(h, None, plan_a, n2, with_counts=True)
    h2 = _pool_div(sums, cnt, bn2)

    for mp in [p['m4'], p['m5']]:
        wihT = mp['Wih'].T
        whhT = mp['Whh'].T
        bih = mp['bih'].reshape(1, -1)
        bhh = mp['bhh'].reshape(1, -1)
        for l in range(mp['W'].shape[0]):
            m = _mm(h2, mp['W'][l], block=bn2)
            agg = _sc_round(m, None, plan2, n2)
            h2 = _gru(agg, h2, wihT, whhT, bih, bhh, l == 2, bn2)

    q2, us2 = _set2set(h2, batch3_2, p['s2s2'], bn2)

    (w0, b0), (w1, b1), (w2, b2) = p['fc']
    (g0, bb0), (g1, bb1) = p['fc_bn']
    return _head(q1, us1, q2, us2,
                 p['pre_bn'][0].reshape(1, -1), p['pre_bn'][1].reshape(1, -1),
                 w0, b0.reshape(1, -1), g0.reshape(1, -1), bb0.reshape(1, -1),
                 w1, b1.reshape(1, -1), g1.reshape(1, -1), bb1.reshape(1, -1),
                 w2, b2.reshape(1, -1))
